# baseline (speedup 1.0000x reference)
import jax
import jax.numpy as jnp
from jax.experimental import pallas as pl
from jax.experimental.pallas import tpu as pltpu

N = 100000
E = 1600000
FX = 32
FE = 16
H = 64
OUT = 64

_BE = 16000
_BN = 2000


def _edge_kernel(xg_ref, ea_ref, w1x_ref, w1e_ref, b1_ref, o_ref):
    h = jnp.dot(xg_ref[...], w1x_ref[...], preferred_element_type=jnp.float32)
    h = h + jnp.dot(ea_ref[...], w1e_ref[...], preferred_element_type=jnp.float32)
    o_ref[...] = jnp.maximum(h + b1_ref[...], 0.0)


def _node_kernel(x_ref, s_ref, c_ref, w2_ref, b2_ref, w3x_ref, w3m_ref,
                 b3_ref, w4_ref, b4_ref, o_ref):
    cnt = c_ref[...]
    mean_relu = s_ref[...] / jnp.maximum(cnt, 1.0)
    m2 = jnp.dot(mean_relu, w2_ref[...], preferred_element_type=jnp.float32) + b2_ref[...]
    m2 = jnp.where(cnt > 0.0, m2, 0.0)
    h = (jnp.dot(x_ref[...], w3x_ref[...], preferred_element_type=jnp.float32)
         + jnp.dot(m2, w3m_ref[...], preferred_element_type=jnp.float32)
         + b3_ref[...])
    h = jnp.maximum(h, 0.0)
    o_ref[...] = jnp.dot(h, w4_ref[...], preferred_element_type=jnp.float32) + b4_ref[...]


def _edge_mlp(xg, ea, W1x, W1e, b1):
    grid = (E // _BE,)
    return pl.pallas_call(
        _edge_kernel,
        grid=grid,
        in_specs=[
            pl.BlockSpec((_BE, FX), lambda i: (i, 0)),
            pl.BlockSpec((_BE, FE), lambda i: (i, 0)),
            pl.BlockSpec((FX, H), lambda i: (0, 0)),
            pl.BlockSpec((FE, H), lambda i: (0, 0)),
            pl.BlockSpec((1, H), lambda i: (0, 0)),
        ],
        out_specs=pl.BlockSpec((_BE, H), lambda i: (i, 0)),
        out_shape=jax.ShapeDtypeStruct((E, H), jnp.float32),
    )(xg, ea, W1x, W1e, b1)


def _node_mlp(x, sums, counts, W2, b2, W3x, W3m, b3, W4, b4):
    grid = (N // _BN,)
    return pl.pallas_call(
        _node_kernel,
        grid=grid,
        in_specs=[
            pl.BlockSpec((_BN, FX), lambda i: (i, 0)),
            pl.BlockSpec((_BN, H), lambda i: (i, 0)),
            pl.BlockSpec((_BN, 1), lambda i: (i, 0)),
            pl.BlockSpec((H, H), lambda i: (0, 0)),
            pl.BlockSpec((1, H), lambda i: (0, 0)),
            pl.BlockSpec((FX, H), lambda i: (0, 0)),
            pl.BlockSpec((H, H), lambda i: (0, 0)),
            pl.BlockSpec((1, H), lambda i: (0, 0)),
            pl.BlockSpec((H, OUT), lambda i: (0, 0)),
            pl.BlockSpec((1, OUT), lambda i: (0, 0)),
        ],
        out_specs=pl.BlockSpec((_BN, OUT), lambda i: (i, 0)),
        out_shape=jax.ShapeDtypeStruct((N, OUT), jnp.float32),
    )(x, sums, counts, W2, b2, W3x, W3m, b3, W4, b4)


def kernel(x, edge_index, edge_attr, u, batch, W1, b1, W2, b2, W3, b3, W4, b4):
    row = edge_index[0]
    col = edge_index[1]
    xg = jnp.take(x, row, axis=0, mode="clip")
    relu_h = _edge_mlp(xg, edge_attr, W1[:FX], W1[FX:], b1.reshape(1, H))
    sums = jax.ops.segment_sum(relu_h, col, num_segments=N)
    counts = jax.ops.segment_sum(jnp.ones((E, 1), jnp.float32), col, num_segments=N)
    return _node_mlp(x, sums, counts, W2, b2.reshape(1, H), W3[:FX], W3[FX:],
                     b3.reshape(1, H), W4, b4.reshape(1, OUT))


# speedup vs baseline: 3.1242x; 3.1242x over previous
import functools

import jax
import jax.numpy as jnp
from jax import lax
from jax.experimental import pallas as pl
from jax.experimental.pallas import tpu as pltpu
from jax.experimental.pallas import tpu_sc as plsc

N = 100000
E = 1600000
FX = 32
FE = 16
H = 64
OUT = 64

_BE = 16000
_BN = 2000

_NC = 2
_NS = 16
_NW = _NC * _NS
_PER_W = E // _NW
_GC = 2000
_GITER = _PER_W // _GC


def _gather_body(row_hbm, x_hbm, out_hbm, idx_v, rows_v, sem):
    wid = lax.axis_index("s") * _NC + lax.axis_index("c")
    wbase = wid * _PER_W

    def body(i, carry):
        base = wbase + i * _GC
        pltpu.sync_copy(row_hbm.at[pl.ds(base, _GC)], idx_v)
        pltpu.async_copy(x_hbm.at[idx_v], rows_v, sem).wait()
        pltpu.sync_copy(rows_v, out_hbm.at[pl.ds(base, _GC)])
        return carry

    lax.fori_loop(0, _GITER, body, 0)


def _sc_gather(row, x):
    mesh = plsc.VectorSubcoreMesh(core_axis_name="c", subcore_axis_name="s")
    k = functools.partial(
        pl.kernel,
        mesh=mesh,
        out_type=jax.ShapeDtypeStruct((E, FX), jnp.float32),
        scratch_types=[
            pltpu.VMEM((_GC,), jnp.int32),
            pltpu.VMEM((_GC, FX), jnp.float32),
            pltpu.SemaphoreType.DMA,
        ],
        compiler_params=pltpu.CompilerParams(use_tc_tiling_on_sc=False),
    )(_gather_body)
    return k(row, x)


_SC_NODES = N // _NC
_ACC_R = 51200
_DUMP_BASE = _SC_NODES
_SCC = 800
_PER_T = E // _NS
_SCITER = _PER_T // _SCC


def _transform_idx(raw_v, idxl_v, lo):
    def tbody(j, carry):
        v = raw_v[pl.ds(j * 16, 16)]
        inr = (v >= lo) & (v < lo + _SC_NODES)
        dump = _DUMP_BASE + jnp.bitwise_and(v, 1023)
        idxl_v[pl.ds(j * 16, 16)] = jnp.where(inr, v - lo, dump)
        return carry

    lax.fori_loop(0, _SCC // 16, tbody, 0)


def _zero_acc(acc, stag, w, sid):
    zero16 = jnp.zeros((16,), jnp.float32)

    def zb(i, carry):
        def zl(l, c2):
            stag[i, pl.ds(l * 16, 16)] = zero16
            return c2
        lax.fori_loop(0, w // 16, zl, 0)
        return carry

    lax.fori_loop(0, _SCC, zb, 0)
    tbase = sid * (_ACC_R // _NS)

    def zc(k, carry):
        pltpu.sync_copy(stag, acc.at[pl.ds(tbase + k * _SCC, _SCC)])
        return carry

    lax.fori_loop(0, _ACC_R // _NS // _SCC, zc, 0)


def _writeout(acc, stag, out_hbm, cid, sid):
    nbase = sid * (_SC_NODES // _NS)
    obase = cid * _SC_NODES + nbase

    def wc(k, carry):
        pltpu.sync_copy(acc.at[pl.ds(nbase + k * 625, 625)], stag.at[pl.ds(0, 625)])
        pltpu.sync_copy(stag.at[pl.ds(0, 625)], out_hbm.at[pl.ds(obase + k * 625, 625)])
        return carry

    lax.fori_loop(0, _SC_NODES // _NS // 625, wc, 0)


def _make_sum_body(co):
    def body(cols_hbm, vals_hbm, out_hbm, raw_v, idxl_v, stag_v, acc):
        cid = lax.axis_index("c")
        sid = lax.axis_index("s")
        lo = cid * _SC_NODES
        _zero_acc(acc, stag_v, 32, sid)
        plsc.subcore_barrier()
        ebase = sid * _PER_T

        def chunk(i, carry):
            base = ebase + i * _SCC
            pltpu.sync_copy(cols_hbm.at[pl.ds(base, _SCC)], raw_v)
            _transform_idx(raw_v, idxl_v, lo)
            pltpu.sync_copy(vals_hbm.at[pl.ds(base, _SCC), pl.ds(co, 32)], stag_v)
            pltpu.sync_copy(stag_v, acc.at[idxl_v], add=True)
            return carry

        lax.fori_loop(0, _SCITER, chunk, 0)
        plsc.subcore_barrier()
        _writeout(acc, stag_v, out_hbm, cid, sid)

    return body


def _count_body(cols_hbm, out_hbm, raw_v, idxl_v, stag_v, acc):
    cid = lax.axis_index("c")
    sid = lax.axis_index("s")
    lo = cid * _SC_NODES
    _zero_acc(acc, stag_v, 16, sid)
    one16 = jnp.where(lax.iota(jnp.int32, 16) == 0, 1.0, 0.0).astype(jnp.float32)

    def sinit(i, carry):
        stag_v[i] = one16
        return carry

    lax.fori_loop(0, _SCC, sinit, 0)
    plsc.subcore_barrier()
    ebase = sid * _PER_T

    def chunk(i, carry):
        base = ebase + i * _SCC
        pltpu.sync_copy(cols_hbm.at[pl.ds(base, _SCC)], raw_v)
        _transform_idx(raw_v, idxl_v, lo)
        pltpu.sync_copy(stag_v, acc.at[idxl_v], add=True)
        return carry

    lax.fori_loop(0, _SCITER, chunk, 0)
    plsc.subcore_barrier()
    _writeout(acc, stag_v, out_hbm, cid, sid)


def _sc_segment_sums(col, relu_h):
    mesh = plsc.VectorSubcoreMesh(core_axis_name="c", subcore_axis_name="s")
    parts = []
    for co in (0, 32):
        k = functools.partial(
            pl.kernel,
            mesh=mesh,
            out_type=jax.ShapeDtypeStruct((N, 32), jnp.float32),
            scratch_types=[
                pltpu.VMEM((_SCC,), jnp.int32),
                pltpu.VMEM((_SCC,), jnp.int32),
                pltpu.VMEM((_SCC, 32), jnp.float32),
                pltpu.VMEM_SHARED((_ACC_R, 32), jnp.float32),
            ],
            compiler_params=pltpu.CompilerParams(use_tc_tiling_on_sc=False),
        )(_make_sum_body(co))
        parts.append(k(col, relu_h))
    kc = functools.partial(
        pl.kernel,
        mesh=mesh,
        out_type=jax.ShapeDtypeStruct((N, 16), jnp.float32),
        scratch_types=[
            pltpu.VMEM((_SCC,), jnp.int32),
            pltpu.VMEM((_SCC,), jnp.int32),
            pltpu.VMEM((_SCC, 16), jnp.float32),
            pltpu.VMEM_SHARED((_ACC_R, 16), jnp.float32),
        ],
        compiler_params=pltpu.CompilerParams(use_tc_tiling_on_sc=False),
    )(_count_body)
    counts = kc(col)
    return parts[0], parts[1], counts


def _edge_kernel(xg_ref, ea_ref, w1x_ref, w1e_ref, b1_ref, o_ref):
    h = jnp.dot(xg_ref[...], w1x_ref[...], preferred_element_type=jnp.float32)
    h = h + jnp.dot(ea_ref[...], w1e_ref[...], preferred_element_type=jnp.float32)
    o_ref[...] = jnp.maximum(h + b1_ref[...], 0.0)


def _node_kernel(x_ref, s0_ref, s1_ref, c_ref, w2_ref, b2_ref, w3x_ref, w3m_ref,
                 b3_ref, w4_ref, b4_ref, o_ref):
    cnt = c_ref[:, 0:1]
    sums = jnp.concatenate([s0_ref[...], s1_ref[...]], axis=1)
    mean_relu = sums / jnp.maximum(cnt, 1.0)
    m2 = jnp.dot(mean_relu, w2_ref[...], preferred_element_type=jnp.float32) + b2_ref[...]
    m2 = jnp.where(cnt > 0.0, m2, 0.0)
    h = (jnp.dot(x_ref[...], w3x_ref[...], preferred_element_type=jnp.float32)
         + jnp.dot(m2, w3m_ref[...], preferred_element_type=jnp.float32)
         + b3_ref[...])
    h = jnp.maximum(h, 0.0)
    o_ref[...] = jnp.dot(h, w4_ref[...], preferred_element_type=jnp.float32) + b4_ref[...]


def _edge_mlp(xg, ea, W1x, W1e, b1):
    grid = (E // _BE,)
    return pl.pallas_call(
        _edge_kernel,
        grid=grid,
        in_specs=[
            pl.BlockSpec((_BE, FX), lambda i: (i, 0)),
            pl.BlockSpec((_BE, FE), lambda i: (i, 0)),
            pl.BlockSpec((FX, H), lambda i: (0, 0)),
            pl.BlockSpec((FE, H), lambda i: (0, 0)),
            pl.BlockSpec((1, H), lambda i: (0, 0)),
        ],
        out_specs=pl.BlockSpec((_BE, H), lambda i: (i, 0)),
        out_shape=jax.ShapeDtypeStruct((E, H), jnp.float32),
    )(xg, ea, W1x, W1e, b1)


def _node_mlp(x, sums0, sums1, counts, W2, b2, W3x, W3m, b3, W4, b4):
    grid = (N // _BN,)
    return pl.pallas_call(
        _node_kernel,
        grid=grid,
        in_specs=[
            pl.BlockSpec((_BN, FX), lambda i: (i, 0)),
            pl.BlockSpec((_BN, 32), lambda i: (i, 0)),
            pl.BlockSpec((_BN, 32), lambda i: (i, 0)),
            pl.BlockSpec((_BN, 16), lambda i: (i, 0)),
            pl.BlockSpec((H, H), lambda i: (0, 0)),
            pl.BlockSpec((1, H), lambda i: (0, 0)),
            pl.BlockSpec((FX, H), lambda i: (0, 0)),
            pl.BlockSpec((H, H), lambda i: (0, 0)),
            pl.BlockSpec((1, H), lambda i: (0, 0)),
            pl.BlockSpec((H, OUT), lambda i: (0, 0)),
            pl.BlockSpec((1, OUT), lambda i: (0, 0)),
        ],
        out_specs=pl.BlockSpec((_BN, OUT), lambda i: (i, 0)),
        out_shape=jax.ShapeDtypeStruct((N, OUT), jnp.float32),
    )(x, sums0, sums1, counts, W2, b2, W3x, W3m, b3, W4, b4)


def kernel(x, edge_index, edge_attr, u, batch, W1, b1, W2, b2, W3, b3, W4, b4):
    row = edge_index[0]
    col = edge_index[1]
    xg = _sc_gather(row, x)
    relu_h = _edge_mlp(xg, edge_attr, W1[:FX], W1[FX:], b1.reshape(1, H))
    sums0, sums1, counts = _sc_segment_sums(col, relu_h)
    return _node_mlp(x, sums0, sums1, counts, W2, b2.reshape(1, H), W3[:FX],
                     W3[FX:], b3.reshape(1, H), W4, b4.reshape(1, OUT))


# speedup vs baseline: 3.5237x; 1.1279x over previous
import functools

import jax
import jax.numpy as jnp
from jax import lax
from jax.experimental import pallas as pl
from jax.experimental.pallas import tpu as pltpu
from jax.experimental.pallas import tpu_sc as plsc

N = 100000
E = 1600000
FX = 32
FE = 16
H = 64
OUT = 64

_BE = 16000
_BN = 2000

_NC = 2
_NS = 16
_NW = _NC * _NS

_GC = 200
_PER_W = E // _NW
_GITER = _PER_W // _GC
_CW = 16


def _gather_counts_body(row_hbm, col_hbm, x_hbm, out_hbm, cnt_hbm,
                        idxb0, idxb1, colb0, colb1, rows0, rows1, stag8, acc,
                        semi0, semi1, semc0, semc1, semg, semo0, semo1):
    idxb = (idxb0, idxb1)
    colb = (colb0, colb1)
    rows = (rows0, rows1)
    semi = (semi0, semi1)
    semc = (semc0, semc1)
    semo = (semo0, semo1)
    cid = lax.axis_index("c")
    sid = lax.axis_index("s")
    wid = sid * _NC + cid
    wbase = wid * _PER_W

    zero16 = jnp.zeros((16,), jnp.float32)

    def zb(i, carry):
        stag8[i, pl.ds(0, 16)] = zero16
        return carry

    lax.fori_loop(0, _GC, zb, 0)
    tbase = sid * (N // _NS)

    def zc(k, carry):
        pltpu.sync_copy(stag8, acc.at[pl.ds(tbase + k * _GC, _GC)])
        return carry

    lax.fori_loop(0, N // _NS // _GC, zc, 0)
    _ztail = N // _NS - (N // _NS // _GC) * _GC
    pltpu.sync_copy(stag8.at[pl.ds(0, _ztail)],
                    acc.at[pl.ds(tbase + N // _NS - _ztail, _ztail)])

    onerow = jnp.where(lax.iota(jnp.int32, 16) == 0, 1.0, 0.0).astype(jnp.float32)

    def ob(i, carry):
        stag8[i, pl.ds(0, 16)] = onerow
        return carry

    lax.fori_loop(0, _GC, ob, 0)
    plsc.subcore_barrier()

    def _issue(i, b):
        pltpu.async_copy(row_hbm.at[pl.ds(wbase + i * _GC, _GC)], idxb[b], semi[b])
        pltpu.async_copy(col_hbm.at[pl.ds(wbase + i * _GC, _GC)], colb[b], semc[b])

    _issue(0, 0)

    def pair(g, carry):
        for b in (0, 1):
            i = g * 2 + b

            @pl.when(i + 1 < _GITER)
            def _():
                _issue(i + 1, 1 - b)

            pltpu.make_async_copy(row_hbm.at[pl.ds(wbase + i * _GC, _GC)],
                                  idxb[b], semi[b]).wait()

            @pl.when(i >= 2)
            def _():
                pltpu.make_async_copy(
                    rows[b], out_hbm.at[pl.ds(wbase + (i - 2) * _GC, _GC)],
                    semo[b]).wait()

            pltpu.async_copy(x_hbm.at[idxb[b]], rows[b], semg).wait()
            pltpu.async_copy(rows[b], out_hbm.at[pl.ds(wbase + i * _GC, _GC)], semo[b])
            pltpu.make_async_copy(col_hbm.at[pl.ds(wbase + i * _GC, _GC)],
                                  colb[b], semc[b]).wait()
            pltpu.sync_copy(stag8, acc.at[colb[b]], add=True)
        return carry

    lax.fori_loop(0, _GITER // 2, pair, 0)
    for b in (0, 1):
        i = _GITER - 2 + b
        pltpu.make_async_copy(rows[b], out_hbm.at[pl.ds(wbase + i * _GC, _GC)],
                              semo[b]).wait()

    plsc.subcore_barrier()
    cbase = cid * N + tbase

    def wc(k, carry):
        pltpu.sync_copy(acc.at[pl.ds(tbase + k * _GC, _GC)], stag8)
        pltpu.sync_copy(stag8, cnt_hbm.at[pl.ds(cbase + k * _GC, _GC)])
        return carry

    lax.fori_loop(0, N // _NS // _GC, wc, 0)
    _wtail = N // _NS - (N // _NS // _GC) * _GC
    _wtb = N // _NS - _wtail
    pltpu.sync_copy(acc.at[pl.ds(tbase + _wtb, _wtail)], stag8.at[pl.ds(0, _wtail)])
    pltpu.sync_copy(stag8.at[pl.ds(0, _wtail)], cnt_hbm.at[pl.ds(cbase + _wtb, _wtail)])


def _sc_gather_counts(row, col, x):
    mesh = plsc.VectorSubcoreMesh(core_axis_name="c", subcore_axis_name="s")
    k = functools.partial(
        pl.kernel,
        mesh=mesh,
        out_type=(
            jax.ShapeDtypeStruct((E, FX), jnp.float32),
            jax.ShapeDtypeStruct((_NC * N, _CW), jnp.float32),
        ),
        scratch_types=[
            pltpu.VMEM((_GC,), jnp.int32),
            pltpu.VMEM((_GC,), jnp.int32),
            pltpu.VMEM((_GC,), jnp.int32),
            pltpu.VMEM((_GC,), jnp.int32),
            pltpu.VMEM((_GC, FX), jnp.float32),
            pltpu.VMEM((_GC, FX), jnp.float32),
            pltpu.VMEM((_GC, _CW), jnp.float32),
            pltpu.VMEM_SHARED((N, _CW), jnp.float32),
            pltpu.SemaphoreType.DMA,
            pltpu.SemaphoreType.DMA,
            pltpu.SemaphoreType.DMA,
            pltpu.SemaphoreType.DMA,
            pltpu.SemaphoreType.DMA,
            pltpu.SemaphoreType.DMA,
            pltpu.SemaphoreType.DMA,
        ],
        compiler_params=pltpu.CompilerParams(use_tc_tiling_on_sc=False),
    )(_gather_counts_body)
    return k(row, col, x)


_SC_NODES = N // _NC
_ACC_R = 51200
_DUMP_BASE = _SC_NODES
_SCC = 400
_PER_T = E // _NS
_SCITER = _PER_T // _SCC


def _transform_idx(raw_v, idxl_v, lo):
    def tbody(j, carry):
        v = raw_v[pl.ds(j * 16, 16)]
        inr = (v >= lo) & (v < lo + _SC_NODES)
        dump = _DUMP_BASE + jnp.bitwise_and(v, 1023)
        idxl_v[pl.ds(j * 16, 16)] = jnp.where(inr, v - lo, dump)
        return carry

    lax.fori_loop(0, _SCC // 16, tbody, 0)


def _make_sum_body(co):
    def body(cols_hbm, vals_hbm, out_hbm, colb0, colb1, stag0, stag1, idxl_v,
             acc, semc0, semc1, sems0, sems1):
        colb = (colb0, colb1)
        stag = (stag0, stag1)
        semc = (semc0, semc1)
        sems = (sems0, sems1)
        cid = lax.axis_index("c")
        sid = lax.axis_index("s")
        lo = cid * _SC_NODES

        zero16 = jnp.zeros((16,), jnp.float32)

        def zb(i, carry):
            stag0[i, pl.ds(0, 16)] = zero16
            stag0[i, pl.ds(16, 16)] = zero16
            return carry

        lax.fori_loop(0, _SCC, zb, 0)
        tbase = sid * (_ACC_R // _NS)

        def zc(k, carry):
            pltpu.sync_copy(stag0, acc.at[pl.ds(tbase + k * _SCC, _SCC)])
            return carry

        lax.fori_loop(0, _ACC_R // _NS // _SCC, zc, 0)
        plsc.subcore_barrier()

        ebase = sid * _PER_T

        def _issue(i, b):
            base = ebase + i * _SCC
            pltpu.async_copy(cols_hbm.at[pl.ds(base, _SCC)], colb[b], semc[b])
            pltpu.async_copy(vals_hbm.at[pl.ds(base, _SCC), pl.ds(co, 32)],
                             stag[b], sems[b])

        _issue(0, 0)

        def pair(g, carry):
            for b in (0, 1):
                i = g * 2 + b
                base = ebase + i * _SCC

                @pl.when(i + 1 < _SCITER)
                def _():
                    _issue(i + 1, 1 - b)

                pltpu.make_async_copy(cols_hbm.at[pl.ds(base, _SCC)],
                                      colb[b], semc[b]).wait()
                _transform_idx(colb[b], idxl_v, lo)
                pltpu.make_async_copy(
                    vals_hbm.at[pl.ds(base, _SCC), pl.ds(co, 32)],
                    stag[b], sems[b]).wait()
                pltpu.sync_copy(stag[b], acc.at[idxl_v], add=True)
            return carry

        lax.fori_loop(0, _SCITER // 2, pair, 0)
        plsc.subcore_barrier()

        nbase = sid * (_SC_NODES // _NS)
        obase = cid * _SC_NODES + nbase

        def wc(k, carry):
            pltpu.sync_copy(acc.at[pl.ds(nbase + k * 125, 125)],
                            stag0.at[pl.ds(0, 125)])
            pltpu.sync_copy(stag0.at[pl.ds(0, 125)],
                            out_hbm.at[pl.ds(obase + k * 125, 125)])
            return carry

        lax.fori_loop(0, _SC_NODES // _NS // 125, wc, 0)

    return body


def _sc_segment_sums(col, relu_h):
    mesh = plsc.VectorSubcoreMesh(core_axis_name="c", subcore_axis_name="s")
    parts = []
    for co in (0, 32):
        k = functools.partial(
            pl.kernel,
            mesh=mesh,
            out_type=jax.ShapeDtypeStruct((N, 32), jnp.float32),
            scratch_types=[
                pltpu.VMEM((_SCC,), jnp.int32),
                pltpu.VMEM((_SCC,), jnp.int32),
                pltpu.VMEM((_SCC, 32), jnp.float32),
                pltpu.VMEM((_SCC, 32), jnp.float32),
                pltpu.VMEM((_SCC,), jnp.int32),
                pltpu.VMEM_SHARED((_ACC_R, 32), jnp.float32),
                pltpu.SemaphoreType.DMA,
                pltpu.SemaphoreType.DMA,
                pltpu.SemaphoreType.DMA,
                pltpu.SemaphoreType.DMA,
            ],
            compiler_params=pltpu.CompilerParams(use_tc_tiling_on_sc=False),
        )(_make_sum_body(co))
        parts.append(k(col, relu_h))
    return parts[0], parts[1]


def _edge_kernel(xg_ref, ea_ref, w1x_ref, w1e_ref, b1_ref, o_ref):
    h = jnp.dot(xg_ref[...], w1x_ref[...], preferred_element_type=jnp.float32)
    h = h + jnp.dot(ea_ref[...], w1e_ref[...], preferred_element_type=jnp.float32)
    o_ref[...] = jnp.maximum(h + b1_ref[...], 0.0)


def _edge_mlp(xg, ea, W1x, W1e, b1):
    grid = (E // _BE,)
    return pl.pallas_call(
        _edge_kernel,
        grid=grid,
        in_specs=[
            pl.BlockSpec((_BE, FX), lambda i: (i, 0)),
            pl.BlockSpec((_BE, FE), lambda i: (i, 0)),
            pl.BlockSpec((FX, H), lambda i: (0, 0)),
            pl.BlockSpec((FE, H), lambda i: (0, 0)),
            pl.BlockSpec((1, H), lambda i: (0, 0)),
        ],
        out_specs=pl.BlockSpec((_BE, H), lambda i: (i, 0)),
        out_shape=jax.ShapeDtypeStruct((E, H), jnp.float32),
    )(xg, ea, W1x, W1e, b1)


def _node_kernel(x_ref, s0_ref, s1_ref, c0_ref, c1_ref, w2_ref, b2_ref,
                 w3x_ref, w3m_ref, b3_ref, w4_ref, b4_ref, o_ref):
    cnt = c0_ref[:, 0:1] + c1_ref[:, 0:1]
    sums = jnp.concatenate([s0_ref[...], s1_ref[...]], axis=1)
    mean_relu = sums / jnp.maximum(cnt, 1.0)
    m2 = jnp.dot(mean_relu, w2_ref[...], preferred_element_type=jnp.float32) + b2_ref[...]
    m2 = jnp.where(cnt > 0.0, m2, 0.0)
    h = (jnp.dot(x_ref[...], w3x_ref[...], preferred_element_type=jnp.float32)
         + jnp.dot(m2, w3m_ref[...], preferred_element_type=jnp.float32)
         + b3_ref[...])
    h = jnp.maximum(h, 0.0)
    o_ref[...] = jnp.dot(h, w4_ref[...], preferred_element_type=jnp.float32) + b4_ref[...]


def _node_mlp(x, sums0, sums1, pcnt, W2, b2, W3x, W3m, b3, W4, b4):
    grid = (N // _BN,)
    return pl.pallas_call(
        _node_kernel,
        grid=grid,
        in_specs=[
            pl.BlockSpec((_BN, FX), lambda i: (i, 0)),
            pl.BlockSpec((_BN, 32), lambda i: (i, 0)),
            pl.BlockSpec((_BN, 32), lambda i: (i, 0)),
            pl.BlockSpec((_BN, _CW), lambda i: (i, 0)),
            pl.BlockSpec((_BN, _CW), lambda i: (N // _BN + i, 0)),
            pl.BlockSpec((H, H), lambda i: (0, 0)),
            pl.BlockSpec((1, H), lambda i: (0, 0)),
            pl.BlockSpec((FX, H), lambda i: (0, 0)),
            pl.BlockSpec((H, H), lambda i: (0, 0)),
            pl.BlockSpec((1, H), lambda i: (0, 0)),
            pl.BlockSpec((H, OUT), lambda i: (0, 0)),
            pl.BlockSpec((1, OUT), lambda i: (0, 0)),
        ],
        out_specs=pl.BlockSpec((_BN, OUT), lambda i: (i, 0)),
        out_shape=jax.ShapeDtypeStruct((N, OUT), jnp.float32),
    )(x, sums0, sums1, pcnt, pcnt, W2, b2, W3x, W3m, b3, W4, b4)


def kernel(x, edge_index, edge_attr, u, batch, W1, b1, W2, b2, W3, b3, W4, b4):
    row = edge_index[0]
    col = edge_index[1]
    xg, pcnt = _sc_gather_counts(row, col, x)
    relu_h = _edge_mlp(xg, edge_attr, W1[:FX], W1[FX:], b1.reshape(1, H))
    sums0, sums1 = _sc_segment_sums(col, relu_h)
    return _node_mlp(x, sums0, sums1, pcnt, W2, b2.reshape(1, H), W3[:FX],
                     W3[FX:], b3.reshape(1, H), W4, b4.reshape(1, OUT))


# speedup vs baseline: 6.3995x; 1.8161x over previous
import functools

import jax
import jax.numpy as jnp
from jax import lax
from jax.experimental import pallas as pl
from jax.experimental.pallas import tpu as pltpu
from jax.experimental.pallas import tpu_sc as plsc

N = 100000
E = 1600000
FX = 32
FE = 16
H = 64
OUT = 64

_BE = 16000
_BN = 2000

_NC = 2
_NS = 16
_NW = _NC * _NS

_GC = 200
_PER_W = E // _NW
_GITER = _PER_W // _GC
_CW = 16


def _gather_counts_body(row_hbm, col_hbm, x_hbm, out_hbm, cnt_hbm,
                        idxb0, idxb1, colb0, colb1, rows0, rows1, stag8, acc,
                        semi0, semi1, semc0, semc1, semg, semo0, semo1):
    idxb = (idxb0, idxb1)
    colb = (colb0, colb1)
    rows = (rows0, rows1)
    semi = (semi0, semi1)
    semc = (semc0, semc1)
    semo = (semo0, semo1)
    cid = lax.axis_index("c")
    sid = lax.axis_index("s")
    wid = sid * _NC + cid
    wbase = wid * _PER_W

    zero16 = jnp.zeros((16,), jnp.float32)

    def zb(i, carry):
        stag8[i, pl.ds(0, 16)] = zero16
        return carry

    lax.fori_loop(0, _GC, zb, 0)
    tbase = sid * (N // _NS)

    def zc(k, carry):
        pltpu.sync_copy(stag8, acc.at[pl.ds(tbase + k * _GC, _GC)])
        return carry

    lax.fori_loop(0, N // _NS // _GC, zc, 0)
    _ztail = N // _NS - (N // _NS // _GC) * _GC
    pltpu.sync_copy(stag8.at[pl.ds(0, _ztail)],
                    acc.at[pl.ds(tbase + N // _NS - _ztail, _ztail)])

    onerow = jnp.where(lax.iota(jnp.int32, 16) == 0, 1.0, 0.0).astype(jnp.float32)

    def ob(i, carry):
        stag8[i, pl.ds(0, 16)] = onerow
        return carry

    lax.fori_loop(0, _GC, ob, 0)
    plsc.subcore_barrier()

    def _issue(i, b):
        pltpu.async_copy(row_hbm.at[pl.ds(wbase + i * _GC, _GC)], idxb[b], semi[b])
        pltpu.async_copy(col_hbm.at[pl.ds(wbase + i * _GC, _GC)], colb[b], semc[b])

    _issue(0, 0)

    def pair(g, carry):
        for b in (0, 1):
            i = g * 2 + b

            @pl.when(i + 1 < _GITER)
            def _():
                _issue(i + 1, 1 - b)

            pltpu.make_async_copy(row_hbm.at[pl.ds(wbase + i * _GC, _GC)],
                                  idxb[b], semi[b]).wait()

            @pl.when(i >= 2)
            def _():
                pltpu.make_async_copy(
                    rows[b], out_hbm.at[pl.ds(wbase + (i - 2) * _GC, _GC)],
                    semo[b]).wait()

            pltpu.async_copy(x_hbm.at[idxb[b]], rows[b], semg).wait()
            pltpu.async_copy(rows[b], out_hbm.at[pl.ds(wbase + i * _GC, _GC)], semo[b])
            pltpu.make_async_copy(col_hbm.at[pl.ds(wbase + i * _GC, _GC)],
                                  colb[b], semc[b]).wait()
            pltpu.sync_copy(stag8, acc.at[colb[b]], add=True)
        return carry

    lax.fori_loop(0, _GITER // 2, pair, 0)
    for b in (0, 1):
        i = _GITER - 2 + b
        pltpu.make_async_copy(rows[b], out_hbm.at[pl.ds(wbase + i * _GC, _GC)],
                              semo[b]).wait()

    plsc.subcore_barrier()
    cbase = cid * N + tbase

    def wc(k, carry):
        pltpu.sync_copy(acc.at[pl.ds(tbase + k * _GC, _GC)], stag8)
        pltpu.sync_copy(stag8, cnt_hbm.at[pl.ds(cbase + k * _GC, _GC)])
        return carry

    lax.fori_loop(0, N // _NS // _GC, wc, 0)
    _wtail = N // _NS - (N // _NS // _GC) * _GC
    _wtb = N // _NS - _wtail
    pltpu.sync_copy(acc.at[pl.ds(tbase + _wtb, _wtail)], stag8.at[pl.ds(0, _wtail)])
    pltpu.sync_copy(stag8.at[pl.ds(0, _wtail)], cnt_hbm.at[pl.ds(cbase + _wtb, _wtail)])


def _sc_gather_counts(row, col, x):
    mesh = plsc.VectorSubcoreMesh(core_axis_name="c", subcore_axis_name="s")
    k = functools.partial(
        pl.kernel,
        mesh=mesh,
        out_type=(
            jax.ShapeDtypeStruct((E, FX), jnp.float32),
            jax.ShapeDtypeStruct((_NC * N, _CW), jnp.float32),
        ),
        scratch_types=[
            pltpu.VMEM((_GC,), jnp.int32),
            pltpu.VMEM((_GC,), jnp.int32),
            pltpu.VMEM((_GC,), jnp.int32),
            pltpu.VMEM((_GC,), jnp.int32),
            pltpu.VMEM((_GC, FX), jnp.float32),
            pltpu.VMEM((_GC, FX), jnp.float32),
            pltpu.VMEM((_GC, _CW), jnp.float32),
            pltpu.VMEM_SHARED((N, _CW), jnp.float32),
            pltpu.SemaphoreType.DMA,
            pltpu.SemaphoreType.DMA,
            pltpu.SemaphoreType.DMA,
            pltpu.SemaphoreType.DMA,
            pltpu.SemaphoreType.DMA,
            pltpu.SemaphoreType.DMA,
            pltpu.SemaphoreType.DMA,
        ],
        compiler_params=pltpu.CompilerParams(use_tc_tiling_on_sc=False),
    )(_gather_counts_body)
    return k(row, col, x)


_SC_NODES = N // _NC
_ACC_R = 51200
_DUMP_BASE = _SC_NODES
_SCC = 400
_PER_T = E // _NS
_SCITER = _PER_T // _SCC


def _transform_idx(raw_v, idxl_v, lo):
    def tbody(j, carry):
        v = raw_v[pl.ds(j * 16, 16)]
        inr = (v >= lo) & (v < lo + _SC_NODES)
        dump = _DUMP_BASE + jnp.bitwise_and(v, 1023)
        idxl_v[pl.ds(j * 16, 16)] = jnp.where(inr, v - lo, dump)
        return carry

    lax.fori_loop(0, _SCC // 16, tbody, 0)


def _sum_body(cols_hbm, vals_hbm, out_hbm, colb0, colb1, stag0, stag1, idxl_v,
              acc, semc0, semc1, sems0, sems1):
    if True:
        colb = (colb0, colb1)
        stag = (stag0, stag1)
        semc = (semc0, semc1)
        sems = (sems0, sems1)
        cid = lax.axis_index("c")
        sid = lax.axis_index("s")
        lo = cid * _SC_NODES

        zero16 = jnp.zeros((16,), jnp.float32)

        def zb(i, carry):
            stag0[i, pl.ds(0, 16)] = zero16
            stag0[i, pl.ds(16, 16)] = zero16
            return carry

        lax.fori_loop(0, _SCC, zb, 0)
        tbase = sid * (_ACC_R // _NS)

        def zc(k, carry):
            pltpu.sync_copy(stag0, acc.at[pl.ds(tbase + k * _SCC, _SCC)])
            return carry

        lax.fori_loop(0, _ACC_R // _NS // _SCC, zc, 0)
        plsc.subcore_barrier()

        ebase = sid * _PER_T

        def _issue(i, b):
            base = ebase + i * _SCC
            pltpu.async_copy(cols_hbm.at[pl.ds(base, _SCC)], colb[b], semc[b])
            pltpu.async_copy(vals_hbm.at[pl.ds(base, _SCC)], stag[b], sems[b])

        _issue(0, 0)

        def pair(g, carry):
            for b in (0, 1):
                i = g * 2 + b
                base = ebase + i * _SCC

                @pl.when(i + 1 < _SCITER)
                def _():
                    _issue(i + 1, 1 - b)

                pltpu.make_async_copy(cols_hbm.at[pl.ds(base, _SCC)],
                                      colb[b], semc[b]).wait()
                _transform_idx(colb[b], idxl_v, lo)
                pltpu.make_async_copy(vals_hbm.at[pl.ds(base, _SCC)],
                                      stag[b], sems[b]).wait()
                pltpu.sync_copy(stag[b], acc.at[idxl_v], add=True)
            return carry

        lax.fori_loop(0, _SCITER // 2, pair, 0)
        plsc.subcore_barrier()

        nbase = sid * (_SC_NODES // _NS)
        obase = cid * _SC_NODES + nbase

        def wc(k, carry):
            pltpu.sync_copy(acc.at[pl.ds(nbase + k * 125, 125)],
                            stag0.at[pl.ds(0, 125)])
            pltpu.sync_copy(stag0.at[pl.ds(0, 125)],
                            out_hbm.at[pl.ds(obase + k * 125, 125)])
            return carry

        lax.fori_loop(0, _SC_NODES // _NS // 125, wc, 0)


def _sc_segment_sums(col, relu_lo, relu_hi):
    mesh = plsc.VectorSubcoreMesh(core_axis_name="c", subcore_axis_name="s")
    parts = []
    for vals in (relu_lo, relu_hi):
        k = functools.partial(
            pl.kernel,
            mesh=mesh,
            out_type=jax.ShapeDtypeStruct((N, 32), jnp.float32),
            scratch_types=[
                pltpu.VMEM((_SCC,), jnp.int32),
                pltpu.VMEM((_SCC,), jnp.int32),
                pltpu.VMEM((_SCC, 32), jnp.float32),
                pltpu.VMEM((_SCC, 32), jnp.float32),
                pltpu.VMEM((_SCC,), jnp.int32),
                pltpu.VMEM_SHARED((_ACC_R, 32), jnp.float32),
                pltpu.SemaphoreType.DMA,
                pltpu.SemaphoreType.DMA,
                pltpu.SemaphoreType.DMA,
                pltpu.SemaphoreType.DMA,
            ],
            compiler_params=pltpu.CompilerParams(use_tc_tiling_on_sc=False),
        )(_sum_body)
        parts.append(k(col, vals))
    return parts[0], parts[1]


def _edge_kernel(xg4_ref, ea4_ref, wxl_ref, wxh_ref, wel_ref, weh_ref,
                 bl_ref, bh_ref, lo_ref, hi_ref):
    xg4 = xg4_ref[...]
    ea4 = ea4_ref[...]
    lo = (jnp.dot(xg4, wxl_ref[...], preferred_element_type=jnp.float32)
          + jnp.dot(ea4, wel_ref[...], preferred_element_type=jnp.float32)
          + bl_ref[...])
    lo_ref[...] = jnp.maximum(lo, 0.0)
    hi = (jnp.dot(xg4, wxh_ref[...], preferred_element_type=jnp.float32)
          + jnp.dot(ea4, weh_ref[...], preferred_element_type=jnp.float32)
          + bh_ref[...])
    hi_ref[...] = jnp.maximum(hi, 0.0)


def _edge_mlp(xg4, ea4, Wxl, Wxh, Wel, Weh, bl, bh):
    grid = (E // _BE,)
    wspec = [
        pl.BlockSpec((128, 128), lambda i: (0, 0)),
        pl.BlockSpec((128, 128), lambda i: (0, 0)),
        pl.BlockSpec((64, 128), lambda i: (0, 0)),
        pl.BlockSpec((64, 128), lambda i: (0, 0)),
        pl.BlockSpec((1, 128), lambda i: (0, 0)),
        pl.BlockSpec((1, 128), lambda i: (0, 0)),
    ]
    return pl.pallas_call(
        _edge_kernel,
        grid=grid,
        in_specs=[
            pl.BlockSpec((_BE // 4, 128), lambda i: (i, 0)),
            pl.BlockSpec((_BE // 4, 64), lambda i: (i, 0)),
        ] + wspec,
        out_specs=[
            pl.BlockSpec((_BE // 4, 128), lambda i: (i, 0)),
            pl.BlockSpec((_BE // 4, 128), lambda i: (i, 0)),
        ],
        out_shape=[
            jax.ShapeDtypeStruct((E // 4, 128), jnp.float32),
            jax.ShapeDtypeStruct((E // 4, 128), jnp.float32),
        ],
    )(xg4, ea4, Wxl, Wxh, Wel, Weh, bl, bh)


def _node_kernel(x_ref, s0_ref, s1_ref, c0_ref, c1_ref, w2_ref, b2_ref,
                 w3x_ref, w3m_ref, b3_ref, w4_ref, b4_ref, o_ref):
    cnt = c0_ref[:, 0:1] + c1_ref[:, 0:1]
    sums = jnp.concatenate([s0_ref[...], s1_ref[...]], axis=1)
    mean_relu = sums / jnp.maximum(cnt, 1.0)
    m2 = jnp.dot(mean_relu, w2_ref[...], preferred_element_type=jnp.float32) + b2_ref[...]
    m2 = jnp.where(cnt > 0.0, m2, 0.0)
    h = (jnp.dot(x_ref[...], w3x_ref[...], preferred_element_type=jnp.float32)
         + jnp.dot(m2, w3m_ref[...], preferred_element_type=jnp.float32)
         + b3_ref[...])
    h = jnp.maximum(h, 0.0)
    o_ref[...] = jnp.dot(h, w4_ref[...], preferred_element_type=jnp.float32) + b4_ref[...]


def _node_mlp(x, sums0, sums1, pcnt, W2, b2, W3x, W3m, b3, W4, b4):
    grid = (N // _BN,)
    return pl.pallas_call(
        _node_kernel,
        grid=grid,
        in_specs=[
            pl.BlockSpec((_BN, FX), lambda i: (i, 0)),
            pl.BlockSpec((_BN, 32), lambda i: (i, 0)),
            pl.BlockSpec((_BN, 32), lambda i: (i, 0)),
            pl.BlockSpec((_BN, _CW), lambda i: (i, 0)),
            pl.BlockSpec((_BN, _CW), lambda i: (N // _BN + i, 0)),
            pl.BlockSpec((H, H), lambda i: (0, 0)),
            pl.BlockSpec((1, H), lambda i: (0, 0)),
            pl.BlockSpec((FX, H), lambda i: (0, 0)),
            pl.BlockSpec((H, H), lambda i: (0, 0)),
            pl.BlockSpec((1, H), lambda i: (0, 0)),
            pl.BlockSpec((H, OUT), lambda i: (0, 0)),
            pl.BlockSpec((1, OUT), lambda i: (0, 0)),
        ],
        out_specs=pl.BlockSpec((_BN, OUT), lambda i: (i, 0)),
        out_shape=jax.ShapeDtypeStruct((N, OUT), jnp.float32),
    )(x, sums0, sums1, pcnt, pcnt, W2, b2, W3x, W3m, b3, W4, b4)


def kernel(x, edge_index, edge_attr, u, batch, W1, b1, W2, b2, W3, b3, W4, b4):
    row = edge_index[0]
    col = edge_index[1]
    xg, pcnt = _sc_gather_counts(row, col, x)
    eye4 = jnp.eye(4, dtype=jnp.float32)
    Wxl = jnp.kron(eye4, W1[:FX, :32])
    Wxh = jnp.kron(eye4, W1[:FX, 32:])
    Wel = jnp.kron(eye4, W1[FX:, :32])
    Weh = jnp.kron(eye4, W1[FX:, 32:])
    bl = jnp.tile(b1[:32], 4).reshape(1, 128)
    bh = jnp.tile(b1[32:], 4).reshape(1, 128)
    relu_lo, relu_hi = _edge_mlp(xg.reshape(E // 4, 128),
                                 edge_attr.reshape(E // 4, 64),
                                 Wxl, Wxh, Wel, Weh, bl, bh)
    sums0, sums1 = _sc_segment_sums(col, relu_lo.reshape(E, 32),
                                    relu_hi.reshape(E, 32))
    return _node_mlp(x, sums0, sums1, pcnt, W2, b2.reshape(1, H), W3[:FX],
                     W3[FX:], b3.reshape(1, H), W4, b4.reshape(1, OUT))
